# baseline (speedup 1.0000x reference)
import functools

import jax
import jax.numpy as jnp
from jax import lax
from jax.experimental import pallas as pl
from jax.experimental.pallas import tpu as pltpu
from jax.experimental.pallas import tpu_sc as plsc

EMBED = 64
SCALE = 8.0
B = 16384
T = 200
ROW_W = T * EMBED
X_W = 2 * T
NW = 32
ROWS_PER_W = B // NW
G = 4
CHUNKS = ROWS_PER_W // G
TOK = G * T
GROUPS = TOK // 16


def _sc_body(x_hbm, tt_hbm, ct_hbm, out_hbm,
             tab_v, combo_v, x_v, out_v, sem_out, sem_x):
    wid = lax.axis_index("s") * 2 + lax.axis_index("c")

    pltpu.sync_copy(tt_hbm.at[pl.ds(EMBED, 2 * EMBED)], tab_v.at[pl.ds(0, 128)])
    pltpu.sync_copy(ct_hbm.at[pl.ds(EMBED, 2 * EMBED)], tab_v.at[pl.ds(128, 128)])
    for i in range(9):
        a, b = i // 3, i % 3
        for dc in range(EMBED // 16):
            off = dc * 16
            if a:
                va = tab_v[pl.ds((a - 1) * 64 + off, 16)] * SCALE
            else:
                va = jnp.zeros((16,), jnp.float32)
            if b:
                vb = tab_v[pl.ds(128 + (b - 1) * 64 + off, 16)] * SCALE
            else:
                vb = jnp.zeros((16,), jnp.float32)
            combo_v[pl.ds(i * 64 + off, 16)] = va + vb

    iota = lax.iota(jnp.int32, 16)
    row0 = wid * ROWS_PER_W

    def chunk_body(it, _):
        base = row0 + it * G
        pltpu.sync_copy(x_hbm.at[pl.ds(base * X_W, G * X_W)], x_v)

        def group_body(g, _):
            t16 = g * 16 + iota
            xe = plsc.load_gather(x_v, [t16 * 2])
            xo = plsc.load_gather(x_v, [t16 * 2 + 1])
            gbase = (xe * 3 + xo) * EMBED
            sbase = t16 * EMBED
            for d in range(EMBED):
                v = plsc.load_gather(combo_v, [gbase + d])
                plsc.store_scatter(out_v, [sbase + d], v)
            return _

        lax.fori_loop(0, GROUPS, group_body, None)
        pltpu.async_copy(out_v, out_hbm.at[pl.ds(base * ROW_W, G * ROW_W)],
                         sem_out).wait()
        return _

    lax.fori_loop(0, CHUNKS, chunk_body, None)


def kernel(x, token_table, chain_table):
    x_flat = x.reshape(B * X_W)
    tt_flat = token_table.reshape(-1)
    ct_flat = chain_table.reshape(-1)
    mesh = plsc.VectorSubcoreMesh(core_axis_name="c", subcore_axis_name="s")
    run = functools.partial(
        pl.kernel,
        mesh=mesh,
        out_type=jax.ShapeDtypeStruct((B * ROW_W,), jnp.float32),
        scratch_types=[
            pltpu.VMEM((256,), jnp.float32),
            pltpu.VMEM((9 * EMBED,), jnp.float32),
            pltpu.VMEM((G * X_W,), jnp.int32),
            pltpu.VMEM((TOK * EMBED,), jnp.float32),
            pltpu.SemaphoreType.DMA,
            pltpu.SemaphoreType.DMA,
        ],
        compiler_params=pltpu.CompilerParams(needs_layout_passes=False),
    )(_sc_body)
    out_flat = run(x_flat, tt_flat, ct_flat)
    return out_flat.reshape(B, T, EMBED)


# speedup vs baseline: 2.6661x; 2.6661x over previous
import functools

import jax
import jax.numpy as jnp
from jax import lax
from jax.experimental import pallas as pl
from jax.experimental.pallas import tpu as pltpu
from jax.experimental.pallas import tpu_sc as plsc

EMBED = 64
SCALE = 8.0
B = 16384
T = 200
NW = 32
PAIRS_TOTAL = B * T // 2
XROWS = PAIRS_TOTAL // 128
OROWS_PER_W = PAIRS_TOTAL // NW
CHUNK = 256
CHUNKS = OROWS_PER_W // CHUNK
XV_ROWS = 200


def _sc_body(xw_hbm, tabs_hbm, out_hbm, combo_hbm,
             tab_v, combo_v, combo2_v, x_v, kv0, kv1, ov0, ov1,
             sg0, sg1, so0, so1):
    wid = lax.axis_index("s") * 2 + lax.axis_index("c")

    pltpu.sync_copy(tabs_hbm, tab_v)
    for i in range(9):
        a, b = i // 3, i % 3
        for dc in range(EMBED // 16):
            off = dc * 16
            if a:
                va = tab_v[a - 1, pl.ds(off, 16)] * SCALE
            else:
                va = jnp.zeros((16,), jnp.float32)
            if b:
                vb = tab_v[b - 1, pl.ds(EMBED + off, 16)] * SCALE
            else:
                vb = jnp.zeros((16,), jnp.float32)
            combo_v[i, pl.ds(off, 16)] = va + vb

    def pair_row(a, carry):
        def pair_col(b, c2):
            row = a * 9 + b
            for dc in range(EMBED // 16):
                off = dc * 16
                combo2_v[row, pl.ds(off, 16)] = combo_v[a, pl.ds(off, 16)]
                combo2_v[row, pl.ds(EMBED + off, 16)] = combo_v[b, pl.ds(off, 16)]
            return c2
        return lax.fori_loop(0, 9, pair_col, carry)

    lax.fori_loop(0, 9, pair_row, None)
    pltpu.sync_copy(combo2_v, combo_hbm)

    row0x = wid * (OROWS_PER_W // 128)
    row0o = wid * OROWS_PER_W
    sg = [sg0, sg1]
    so = [so0, so1]
    kv = [kv0, kv1]
    ov = [ov0, ov1]

    def out_copy(c, buf):
        return pltpu.make_async_copy(
            ov[buf], out_hbm.at[pl.ds(row0o + c * CHUNK, CHUNK), :], so[buf])

    def pair_body(p, carry):
        for buf in range(2):
            c = p * 2 + buf

            @pl.when(jnp.logical_or(c == 0, c == 100))
            def _load_x():
                pltpu.sync_copy(
                    xw_hbm.at[pl.ds(row0x + jnp.where(c >= 100, XV_ROWS, 0),
                                    XV_ROWS), :],
                    x_v)

            lrow = c * 2 - jnp.where(c >= 100, 200, 0)

            def g_body(g, _):
                w = x_v[lrow + (g >> 3), pl.ds((g & 7) * 16, 16)]
                kk = ((w & 255) * 27 + ((w >> 8) & 255) * 9
                      + ((w >> 16) & 255) * 3 + (w >> 24))
                kv[buf][g >> 3, pl.ds((g & 7) * 16, 16)] = kk
                return _

            lax.fori_loop(0, 16, g_body, None)

            @pl.when(p > 0)
            def _wait_out():
                out_copy(c - 2, buf).wait()

            for r in range(2):
                pltpu.make_async_copy(
                    combo_hbm.at[kv[buf].at[r]],
                    ov[buf].at[pl.ds(r * 128, 128), :],
                    sg[buf]).start()

        for buf in range(2):
            c = p * 2 + buf
            for r in range(2):
                pltpu.make_async_copy(
                    combo_hbm.at[kv[buf].at[r]],
                    ov[buf].at[pl.ds(r * 128, 128), :],
                    sg[buf]).wait()
            out_copy(c, buf).start()
        return carry

    lax.fori_loop(0, CHUNKS // 2, pair_body, None)
    out_copy(CHUNKS - 2, 0).wait()
    out_copy(CHUNKS - 1, 1).wait()


def kernel(x, token_table, chain_table):
    xw = lax.bitcast_convert_type(
        x.astype(jnp.int8).reshape(PAIRS_TOTAL, 4), jnp.int32)
    xw = xw.reshape(XROWS, 128)
    tabs = jnp.concatenate(
        [jnp.concatenate([token_table[1:2], chain_table[1:2]], axis=1),
         jnp.concatenate([token_table[2:3], chain_table[2:3]], axis=1)],
        axis=0)
    mesh = plsc.VectorSubcoreMesh(core_axis_name="c", subcore_axis_name="s")
    run = functools.partial(
        pl.kernel,
        mesh=mesh,
        out_type=[
            jax.ShapeDtypeStruct((PAIRS_TOTAL, 128), jnp.float32),
            jax.ShapeDtypeStruct((88, 128), jnp.float32),
        ],
        scratch_types=[
            pltpu.VMEM((2, 128), jnp.float32),
            pltpu.VMEM((9, EMBED), jnp.float32),
            pltpu.VMEM((88, 128), jnp.float32),
            pltpu.VMEM((XV_ROWS, 128), jnp.int32),
            pltpu.VMEM((2, 128), jnp.int32),
            pltpu.VMEM((2, 128), jnp.int32),
            pltpu.VMEM((CHUNK, 128), jnp.float32),
            pltpu.VMEM((CHUNK, 128), jnp.float32),
            pltpu.SemaphoreType.DMA,
            pltpu.SemaphoreType.DMA,
            pltpu.SemaphoreType.DMA,
            pltpu.SemaphoreType.DMA,
        ],
        compiler_params=pltpu.CompilerParams(
            needs_layout_passes=False, use_tc_tiling_on_sc=True),
    )(_sc_body)
    out_flat, _ = run(xw, tabs)
    return out_flat.reshape(B, T, EMBED)


# speedup vs baseline: 4.1115x; 1.5421x over previous
import functools

import jax
import jax.numpy as jnp
from jax import lax
from jax.experimental import pallas as pl
from jax.experimental.pallas import tpu as pltpu
from jax.experimental.pallas import tpu_sc as plsc

EMBED = 64
SCALE = 8.0
B = 16384
T = 200
NW = 32
PAIRS_TOTAL = B * T // 2
XROWS = PAIRS_TOTAL // 128
OROWS_PER_W = PAIRS_TOTAL // NW
CHUNK = 256
CHUNKS = OROWS_PER_W // CHUNK
XV_ROWS = 200


def _sc_body(xw_hbm, tabs_hbm, out_hbm, combo_hbm,
             tab_v, combo_v, combo2_v, x_v, kv0, kv1, ov0, ov1,
             sg0, sg1, so0, so1):
    wid = lax.axis_index("s") * 2 + lax.axis_index("c")

    pltpu.sync_copy(tabs_hbm, tab_v)
    for i in range(9):
        a, b = i // 3, i % 3
        for dc in range(EMBED // 16):
            off = dc * 16
            if a:
                va = tab_v[a - 1, pl.ds(off, 16)] * SCALE
            else:
                va = jnp.zeros((16,), jnp.float32)
            if b:
                vb = tab_v[b - 1, pl.ds(EMBED + off, 16)] * SCALE
            else:
                vb = jnp.zeros((16,), jnp.float32)
            combo_v[i, pl.ds(off, 16)] = va + vb

    def pair_row(a, carry):
        def pair_col(b, c2):
            row = a * 9 + b
            for dc in range(EMBED // 16):
                off = dc * 16
                combo2_v[row, pl.ds(off, 16)] = combo_v[a, pl.ds(off, 16)]
                combo2_v[row, pl.ds(EMBED + off, 16)] = combo_v[b, pl.ds(off, 16)]
            return c2
        return lax.fori_loop(0, 9, pair_col, carry)

    lax.fori_loop(0, 9, pair_row, None)
    pltpu.sync_copy(combo2_v, combo_hbm.at[pl.ds(wid * 88, 88), :])

    row0x = wid * (OROWS_PER_W // 128)
    row0o = wid * OROWS_PER_W
    sg = [sg0, sg1]
    so = [so0, so1]
    kv = [kv0, kv1]
    ov = [ov0, ov1]

    def out_copy(c, buf):
        return pltpu.make_async_copy(
            ov[buf], out_hbm.at[pl.ds(row0o + c * CHUNK, CHUNK), :], so[buf])

    def pair_body(p, carry):
        for buf in range(2):
            c = p * 2 + buf

            @pl.when(jnp.logical_or(c == 0, c == 100))
            def _load_x():
                pltpu.sync_copy(
                    xw_hbm.at[pl.ds(row0x + jnp.where(c >= 100, XV_ROWS, 0),
                                    XV_ROWS), :],
                    x_v)

            lrow = c * 2 - jnp.where(c >= 100, 200, 0)

            def g_body(g, _):
                w = x_v[lrow + (g >> 3), pl.ds((g & 7) * 16, 16)]
                kk = ((w & 255) * 27 + ((w >> 8) & 255) * 9
                      + ((w >> 16) & 255) * 3 + (w >> 24) + wid * 88)
                kv[buf][g >> 3, pl.ds((g & 7) * 16, 16)] = kk
                return _

            lax.fori_loop(0, 16, g_body, None)

            @pl.when(p > 0)
            def _wait_out():
                out_copy(c - 2, buf).wait()

            for r in range(2):
                pltpu.make_async_copy(
                    combo_hbm.at[kv[buf].at[r]],
                    ov[buf].at[pl.ds(r * 128, 128), :],
                    sg[buf]).start()

        for buf in range(2):
            c = p * 2 + buf
            for r in range(2):
                pltpu.make_async_copy(
                    combo_hbm.at[kv[buf].at[r]],
                    ov[buf].at[pl.ds(r * 128, 128), :],
                    sg[buf]).wait()
            out_copy(c, buf).start()
        return carry

    lax.fori_loop(0, CHUNKS // 2, pair_body, None)
    out_copy(CHUNKS - 2, 0).wait()
    out_copy(CHUNKS - 1, 1).wait()


def kernel(x, token_table, chain_table):
    xw = lax.bitcast_convert_type(
        x.astype(jnp.int8).reshape(PAIRS_TOTAL, 4), jnp.int32)
    xw = xw.reshape(XROWS, 128)
    tabs = jnp.concatenate(
        [jnp.concatenate([token_table[1:2], chain_table[1:2]], axis=1),
         jnp.concatenate([token_table[2:3], chain_table[2:3]], axis=1)],
        axis=0)
    mesh = plsc.VectorSubcoreMesh(core_axis_name="c", subcore_axis_name="s")
    run = functools.partial(
        pl.kernel,
        mesh=mesh,
        out_type=[
            jax.ShapeDtypeStruct((PAIRS_TOTAL, 128), jnp.float32),
            jax.ShapeDtypeStruct((NW * 88, 128), jnp.float32),
        ],
        scratch_types=[
            pltpu.VMEM((2, 128), jnp.float32),
            pltpu.VMEM((9, EMBED), jnp.float32),
            pltpu.VMEM((88, 128), jnp.float32),
            pltpu.VMEM((XV_ROWS, 128), jnp.int32),
            pltpu.VMEM((2, 128), jnp.int32),
            pltpu.VMEM((2, 128), jnp.int32),
            pltpu.VMEM((CHUNK, 128), jnp.float32),
            pltpu.VMEM((CHUNK, 128), jnp.float32),
            pltpu.SemaphoreType.DMA,
            pltpu.SemaphoreType.DMA,
            pltpu.SemaphoreType.DMA,
            pltpu.SemaphoreType.DMA,
        ],
        compiler_params=pltpu.CompilerParams(
            needs_layout_passes=False, use_tc_tiling_on_sc=True),
    )(_sc_body)
    out_flat, _ = run(xw, tabs)
    return out_flat.reshape(B, T, EMBED)
